# baseline (speedup 1.0000x reference)
import functools

import jax
import jax.numpy as jnp
from jax import lax
from jax.experimental import pallas as pl
from jax.experimental.pallas import tpu as pltpu, tpu_sc as plsc

_NW = 32
_CB = 2000
_L = 16


@functools.cache
def _build(n, m):
    num_chunks = n // _CB
    groups = _CB // _L
    mesh = plsc.VectorSubcoreMesh(core_axis_name="c", subcore_axis_name="s")

    @functools.partial(
        pl.kernel,
        out_type=(
            jax.ShapeDtypeStruct((n, 3), jnp.float32),
            jax.ShapeDtypeStruct((n, 3), jnp.float32),
        ),
        mesh=mesh,
        scratch_types=[
            pltpu.VMEM((m, 4, 4), jnp.float32),
            pltpu.VMEM((_CB, 3), jnp.float32),
            pltpu.VMEM((_CB, 3), jnp.float32),
            pltpu.VMEM((_CB, 4), jnp.float32),
            pltpu.VMEM((_CB, 4), jnp.int32),
            pltpu.VMEM((_CB, 3), jnp.float32),
            pltpu.VMEM((_CB, 3), jnp.float32),
        ],
        compiler_params=pltpu.CompilerParams(
            use_tc_tiling_on_sc=False, needs_layout_passes=False),
    )
    def skin(v_hbm, nrm_hbm, w_hbm, idx_hbm, tab_hbm,
             ov_hbm, on_hbm,
             tab_v, v_v, n_v, w_v, i_v, ov_v, on_v):
        cid = lax.axis_index("c")
        sid = lax.axis_index("s")
        wid = sid * 2 + cid

        pltpu.sync_copy(tab_hbm, tab_v)

        lane = lax.iota(jnp.int32, _L)
        zero = jnp.zeros((_L,), jnp.float32)
        col = [jnp.full((_L,), d, jnp.int32) for d in range(4)]

        def group(g, carry):
            rows = lane + g * _L
            x = plsc.load_gather(v_v, [rows, col[0]])
            y = plsc.load_gather(v_v, [rows, col[1]])
            z = plsc.load_gather(v_v, [rows, col[2]])
            nx = plsc.load_gather(n_v, [rows, col[0]])
            ny = plsc.load_gather(n_v, [rows, col[1]])
            nz = plsc.load_gather(n_v, [rows, col[2]])
            av0 = av1 = av2 = zero
            an0 = an1 = an2 = zero
            for i in range(4):
                bi = plsc.load_gather(i_v, [rows, col[i]])
                w = plsc.load_gather(w_v, [rows, col[i]])
                mm = [plsc.load_gather(tab_v, [bi, col[k // 4], col[k % 4]])
                      for k in range(16)]
                t0 = x * mm[0] + y * mm[1] + z * mm[2] + mm[3]
                t1 = x * mm[4] + y * mm[5] + z * mm[6] + mm[7]
                t2 = x * mm[8] + y * mm[9] + z * mm[10] + mm[11]
                t3 = x * mm[12] + y * mm[13] + z * mm[14] + mm[15]
                r = w / t3
                av0 = av0 + t0 * r
                av1 = av1 + t1 * r
                av2 = av2 + t2 * r
                an0 = an0 + w * (nx * mm[0] + ny * mm[1] + nz * mm[2])
                an1 = an1 + w * (nx * mm[4] + ny * mm[5] + nz * mm[6])
                an2 = an2 + w * (nx * mm[8] + ny * mm[9] + nz * mm[10])
            plsc.store_scatter(ov_v, [rows, col[0]], av0)
            plsc.store_scatter(ov_v, [rows, col[1]], av1)
            plsc.store_scatter(ov_v, [rows, col[2]], av2)
            plsc.store_scatter(on_v, [rows, col[0]], an0)
            plsc.store_scatter(on_v, [rows, col[1]], an1)
            plsc.store_scatter(on_v, [rows, col[2]], an2)
            return carry

        def chunk(ci, carry):
            c = wid + ci * _NW
            b = c * _CB
            pltpu.sync_copy(v_hbm.at[pl.ds(b, _CB)], v_v)
            pltpu.sync_copy(nrm_hbm.at[pl.ds(b, _CB)], n_v)
            pltpu.sync_copy(w_hbm.at[pl.ds(b, _CB)], w_v)
            pltpu.sync_copy(idx_hbm.at[pl.ds(b, _CB)], i_v)
            lax.fori_loop(0, groups, group, 0, unroll=False)
            pltpu.sync_copy(ov_v, ov_hbm.at[pl.ds(b, _CB)])
            pltpu.sync_copy(on_v, on_hbm.at[pl.ds(b, _CB)])
            return carry

        my_chunks = (num_chunks - 1 - wid) // _NW + 1
        lax.fori_loop(0, my_chunks, chunk, 0, unroll=False)

    return skin


def kernel(vertices, normals, bone_weights, bone_indices, bone_matrices):
    n = vertices.shape[0]
    m = bone_matrices.shape[0]
    pad = (-n) % _CB
    if pad:
        vertices = jnp.pad(vertices, ((0, pad), (0, 0)))
        normals = jnp.pad(normals, ((0, pad), (0, 0)))
        bone_weights = jnp.pad(bone_weights, ((0, pad), (0, 0)))
        bone_indices = jnp.pad(bone_indices, ((0, pad), (0, 0)))
    idx32 = bone_indices.astype(jnp.int32)
    ov, on = _build(n + pad, m)(
        vertices.astype(jnp.float32),
        normals.astype(jnp.float32),
        bone_weights.astype(jnp.float32),
        idx32,
        bone_matrices.astype(jnp.float32),
    )
    return (ov[:n], on[:n]) if pad else (ov, on)


# speedup vs baseline: 8.8249x; 8.8249x over previous
import functools

import jax
import jax.numpy as jnp
from jax import lax
from jax.experimental import pallas as pl
from jax.experimental.pallas import tpu as pltpu, tpu_sc as plsc

_NW = 32
_CB = 2000
_L = 16


@functools.cache
def _build(n, m):
    num_chunks = n // _CB
    groups = _CB // _L
    mesh = plsc.VectorSubcoreMesh(core_axis_name="c", subcore_axis_name="s")
    f32 = jnp.float32

    @functools.partial(
        pl.kernel,
        out_type=tuple(jax.ShapeDtypeStruct((n,), f32) for _ in range(6)),
        mesh=mesh,
        scratch_types=[
            pltpu.VMEM((m, 4, 4), f32),
        ] + [pltpu.VMEM((_CB,), f32) for _ in range(6)]
          + [pltpu.VMEM((_CB,), f32) for _ in range(4)]
          + [pltpu.VMEM((_CB,), jnp.int32) for _ in range(4)]
          + [pltpu.VMEM((_CB,), f32) for _ in range(6)]
          + [pltpu.SemaphoreType.DMA],
        compiler_params=pltpu.CompilerParams(
            use_tc_tiling_on_sc=False, needs_layout_passes=False),
    )
    def skin(*refs):
        ins = refs[:15]
        outs = refs[15:21]
        tab_v = refs[21]
        in_v = refs[22:36]
        out_v = refs[36:42]
        sem = refs[42]

        cid = lax.axis_index("c")
        sid = lax.axis_index("s")
        wid = sid * 2 + cid

        pltpu.sync_copy(ins[14], tab_v)

        col = [jnp.full((_L,), d, jnp.int32) for d in range(4)]

        def group(g, carry):
            s = pl.ds(g * _L, _L)
            x, y, z = in_v[0][s], in_v[1][s], in_v[2][s]
            nx, ny, nz = in_v[3][s], in_v[4][s], in_v[5][s]
            av0 = av1 = av2 = jnp.zeros((_L,), f32)
            an0 = an1 = an2 = jnp.zeros((_L,), f32)
            for i in range(4):
                w = in_v[6 + i][s]
                bi = in_v[10 + i][s]
                mm = [plsc.load_gather(tab_v, [bi, col[k // 4], col[k % 4]])
                      for k in range(16)]
                t0 = x * mm[0] + y * mm[1] + z * mm[2] + mm[3]
                t1 = x * mm[4] + y * mm[5] + z * mm[6] + mm[7]
                t2 = x * mm[8] + y * mm[9] + z * mm[10] + mm[11]
                t3 = x * mm[12] + y * mm[13] + z * mm[14] + mm[15]
                r = w / t3
                av0 = av0 + t0 * r
                av1 = av1 + t1 * r
                av2 = av2 + t2 * r
                an0 = an0 + w * (nx * mm[0] + ny * mm[1] + nz * mm[2])
                an1 = an1 + w * (nx * mm[4] + ny * mm[5] + nz * mm[6])
                an2 = an2 + w * (nx * mm[8] + ny * mm[9] + nz * mm[10])
            out_v[0][s], out_v[1][s], out_v[2][s] = av0, av1, av2
            out_v[3][s], out_v[4][s], out_v[5][s] = an0, an1, an2
            return carry

        def chunk(ci, carry):
            c = wid + ci * _NW
            b = c * _CB
            cps = [pltpu.make_async_copy(ins[j].at[pl.ds(b, _CB)], in_v[j], sem)
                   for j in range(14)]
            for cp in cps:
                cp.start()
            for cp in cps:
                cp.wait()
            lax.fori_loop(0, groups, group, 0, unroll=False)
            ocs = [pltpu.make_async_copy(out_v[j], outs[j].at[pl.ds(b, _CB)], sem)
                   for j in range(6)]
            for oc in ocs:
                oc.start()
            for oc in ocs:
                oc.wait()
            return carry

        my_chunks = (num_chunks - 1 - wid) // _NW + 1
        lax.fori_loop(0, my_chunks, chunk, 0, unroll=False)

    return skin


def kernel(vertices, normals, bone_weights, bone_indices, bone_matrices):
    n = vertices.shape[0]
    m = bone_matrices.shape[0]
    pad = (-n) % _CB
    if pad:
        vertices = jnp.pad(vertices, ((0, pad), (0, 0)))
        normals = jnp.pad(normals, ((0, pad), (0, 0)))
        bone_weights = jnp.pad(bone_weights, ((0, pad), (0, 0)))
        bone_indices = jnp.pad(bone_indices, ((0, pad), (0, 0)))
    vertices = vertices.astype(jnp.float32)
    normals = normals.astype(jnp.float32)
    bone_weights = bone_weights.astype(jnp.float32)
    idx32 = bone_indices.astype(jnp.int32)
    planes = (
        [vertices[:, d] for d in range(3)]
        + [normals[:, d] for d in range(3)]
        + [bone_weights[:, d] for d in range(4)]
        + [idx32[:, d] for d in range(4)]
    )
    outs = _build(n + pad, m)(*planes, bone_matrices.astype(jnp.float32))
    ov = jnp.stack(outs[:3], axis=1)[:n]
    on = jnp.stack(outs[3:], axis=1)[:n]
    return ov, on
